# baseline (speedup 1.0000x reference)
import functools

import jax
import jax.numpy as jnp
import numpy as np
from jax import lax
from jax.experimental import pallas as pl
from jax.experimental.pallas import tpu as pltpu
from jax.experimental.pallas import tpu_sc as plsc

C1 = 1e-6
C2 = 1e-6
NW = 32
G = 8
F0 = 0.975


BW = 512


def _build_body(x_ref, xp_ref, t_ref, m_ref, v_ref):
    c = x_ref.shape[0]

    def pack_half(ref):
        xt = ref[...].T
        lo = jnp.concatenate(
            [xt[:, u * 32:u * 32 + 16] for u in range(c // 32)], axis=1)
        hi = jnp.concatenate(
            [xt[:, u * 32 + 16:u * 32 + 32] for u in range(c // 32)], axis=1)
        lo_w = lax.convert_element_type(
            lax.bitcast_convert_type(lo.astype(jnp.bfloat16), jnp.uint16),
            jnp.uint32)
        hi_w = lax.convert_element_type(
            lax.bitcast_convert_type(hi.astype(jnp.bfloat16), jnp.uint16),
            jnp.uint32)
        return lo_w | (hi_w << 16)

    xb = x_ref[...]
    m = jnp.mean(xb, axis=0, keepdims=True)
    q = jnp.mean(xb * xb, axis=0, keepdims=True)
    m_ref[...] = m
    v_ref[...] = q - m * m
    t_ref[...] = jnp.concatenate([pack_half(x_ref), pack_half(xp_ref)], axis=1)


def _make_sc_kernel(n_pad, c, k, npw0, npw1):
    ng0 = npw0 // G
    ng1 = npw1 // G
    ng = max(ng0, ng1)
    ge = G * k
    cc = c // 16
    c32 = c // 32
    mesh = plsc.VectorSubcoreMesh(core_axis_name="c", subcore_axis_name="s",
                                  num_cores=2, num_subcores=16)

    @functools.partial(
        pl.kernel,
        out_type=jax.ShapeDtypeStruct((n_pad, c), jnp.float32),
        mesh=mesh,
        compiler_params=pltpu.CompilerParams(needs_layout_passes=False),
        scratch_types=[
            pltpu.VMEM((n_pad,), jnp.float32),
            pltpu.VMEM((n_pad,), jnp.float32),
            pltpu.VMEM((ng, ge), jnp.int32),
            pltpu.VMEM((ng, ge), jnp.int32),
            pltpu.VMEM((ge, c), jnp.uint32),
            pltpu.VMEM((ge, c), jnp.uint32),
            pltpu.VMEM((ge, c), jnp.uint32),
            pltpu.VMEM((ge, c), jnp.uint32),
            pltpu.VMEM((G, c), jnp.float32),
            pltpu.VMEM((G, c), jnp.float32),
            pltpu.VMEM_SHARED((n_pad,), jnp.float32),
            pltpu.VMEM_SHARED((n_pad,), jnp.float32),
            pltpu.SemaphoreType.DMA,
            pltpu.SemaphoreType.DMA,
            pltpu.SemaphoreType.DMA,
            pltpu.SemaphoreType.DMA,
            pltpu.SemaphoreType.DMA,
            pltpu.SemaphoreType.DMA,
        ],
    )
    def sc_kernel(xall_h, eii_h, eij_h, m_h, v_h, out_h,
                  m_v, v_v, eii_v, eij_v,
                  ibuf0, ibuf1, jbuf0, jbuf1, obuf0, obuf1, m_sh, v_sh,
                  si0, si1, sj0, sj1, so0, so1):
        ci = lax.axis_index("c")
        si = lax.axis_index("s")
        wid = ci * 16 + si
        node0 = jnp.where(ci == 0, si * npw0, 16 * npw0 + si * npw1)
        ngw = jnp.where(ci == 0, ng0, ng1)

        with jax.named_scope("sc_prologue"):
            @pl.when(si == 0)
            def _():
                pltpu.sync_copy(m_h, m_sh)
                pltpu.sync_copy(v_h, v_sh)

            @pl.when(ci == 0)
            def _():
                pltpu.sync_copy(eii_h.at[wid, pl.ds(0, ng0)],
                                eii_v.at[pl.ds(0, ng0)])
                pltpu.sync_copy(eij_h.at[wid, pl.ds(0, ng0)],
                                eij_v.at[pl.ds(0, ng0)])

            @pl.when(ci == 1)
            def _():
                pltpu.sync_copy(eii_h.at[wid, pl.ds(0, ng1)],
                                eii_v.at[pl.ds(0, ng1)])
                pltpu.sync_copy(eij_h.at[wid, pl.ds(0, ng1)],
                                eij_v.at[pl.ds(0, ng1)])

            plsc.subcore_barrier()
            pltpu.sync_copy(m_sh, m_v)
            pltpu.sync_copy(v_sh, v_v)

        maskh = jnp.full((16,), 0xFFFF0000, jnp.uint32)

        def halves(ref, row, half, u):
            w = ref[row, pl.ds(half * (c // 2) + u * 16, 16)]
            lo = plsc.bitcast(w << 16, jnp.float32)
            hi = plsc.bitcast(w & maskh, jnp.float32)
            return lo, hi

        def start(g, ib, jb, si, sj):
            pltpu.async_copy(xall_h.at[eii_v.at[g]], ib, si)
            pltpu.async_copy(xall_h.at[eij_v.at[g]], jb, sj)

        def wait(g, ib, jb, si, sj):
            pltpu.make_async_copy(xall_h.at[eii_v.at[g]], ib, si).wait()
            pltpu.make_async_copy(xall_h.at[eij_v.at[g]], jb, sj).wait()

        def process(g, ib, jb, ob, so):
            @pl.when(g >= 2)
            def _():
                pltpu.make_async_copy(
                    ob, out_h.at[pl.ds(node0 + (g - 2) * G, G)], so).wait()

            def node_body(t, carry):
                row0 = t * k
                iidx = eii_v[g, pl.ds(row0, 16)]
                jidx = eij_v[g, pl.ds(row0, 16)]
                mi = plsc.load_gather(m_v, [iidx])
                mj = plsc.load_gather(m_v, [jidx])
                vi = plsc.load_gather(v_v, [iidx])
                vj = plsc.load_gather(v_v, [jidx])
                mmv = mi * mj
                s1v = (2.0 * mmv + C1) / (mi * mi + mj * mj + C1)
                s1dv = s1v / (vi + vj + C2)
                accs = [jnp.zeros((16,), jnp.float32) for _ in range(cc)]
                for e in range(k):
                    row = row0 + e
                    d0 = None
                    d1 = None
                    for u in range(c32):
                        alo, ahi = halves(ib, row, 0, u)
                        blo, bhi = halves(jb, row, 0, u)
                        p0 = alo * blo
                        p1 = ahi * bhi
                        d0 = p0 if d0 is None else d0 + p0
                        d1 = p1 if d1 is None else d1 + p1
                    dot = jnp.sum(d0 + d1)
                    cov2 = 2.0 * (dot * (1.0 / c) - mmv[e]) + C2
                    sff = jnp.full((16,), 1.0 - s1dv[e] * cov2, jnp.float32)
                    for u in range(c32):
                        alo, ahi = halves(ib, row, 1, u)
                        blo, bhi = halves(jb, row, 1, u)
                        accs[2 * u] = (accs[2 * u] + (alo + blo)
                                       + jnp.abs(alo - blo) * sff)
                        accs[2 * u + 1] = (accs[2 * u + 1] + (ahi + bhi)
                                           + jnp.abs(ahi - bhi) * sff)
                for q in range(cc):
                    ob[t, pl.ds(q * 16, 16)] = accs[q]
                return carry

            lax.fori_loop(0, G, node_body, 0)
            pltpu.async_copy(ob, out_h.at[pl.ds(node0 + g * G, G)], so)

        with jax.named_scope("sc_first_gather"):
            start(0, ibuf0, jbuf0, si0, sj0)

        def outer(tt, carry):
            g0 = 2 * tt
            start(g0 + 1, ibuf1, jbuf1, si1, sj1)
            wait(g0, ibuf0, jbuf0, si0, sj0)
            process(g0, ibuf0, jbuf0, obuf0, so0)
            g1 = g0 + 1

            @pl.when(g1 + 1 < ngw)
            def _():
                start(g1 + 1, ibuf0, jbuf0, si0, sj0)

            wait(g1, ibuf1, jbuf1, si1, sj1)
            process(g1, ibuf1, jbuf1, obuf1, so1)
            return carry

        with jax.named_scope("sc_mainloop"):
            lax.fori_loop(0, ngw // 2, outer, 0)
            pltpu.make_async_copy(
                obuf0, out_h.at[pl.ds(node0 + (ngw - 2) * G, G)], so0).wait()
            pltpu.make_async_copy(
                obuf1, out_h.at[pl.ds(node0 + (ngw - 1) * G, G)], so1).wait()

    return sc_kernel


def _split_edges(ei2, npw0, npw1, k):
    ng0, ng1 = npw0 // G, npw1 // G
    ng = max(ng0, ng1)
    ge = G * k
    cut = 16 * npw0
    p0 = ei2[:cut].reshape(16, ng0, ge)
    p0 = jnp.pad(p0, ((0, 0), (0, ng - ng0), (0, 0)))
    p1 = ei2[cut:].reshape(16, ng1, ge)
    p1 = jnp.pad(p1, ((0, 0), (0, ng - ng1), (0, 0)))
    return jnp.concatenate([p0, p1], axis=0)


def kernel(x, x_p, edge_index):
    b, c, n, _ = x.shape
    k = edge_index.shape[3]
    npw = -(-n // (NW * 2 * G)) * 2 * G
    n_pad = NW * npw
    npw0 = max(2 * G, (int(2 * npw * F0) // (2 * G)) * 2 * G)
    npw1 = 2 * npw - npw0

    x2 = x[0, :, :, 0]
    xp2 = x_p[0, :, :, 0]

    xall, m2, v2 = pl.pallas_call(
        _build_body,
        grid=(n_pad // BW,),
        in_specs=[pl.BlockSpec((c, BW), lambda i: (0, i)),
                  pl.BlockSpec((c, BW), lambda i: (0, i))],
        out_specs=(pl.BlockSpec((BW, c), lambda i: (i, 0)),
                   pl.BlockSpec((1, BW), lambda i: (0, i)),
                   pl.BlockSpec((1, BW), lambda i: (0, i))),
        out_shape=(jax.ShapeDtypeStruct((n_pad, c), jnp.uint32),
                   jax.ShapeDtypeStruct((1, n_pad), jnp.float32),
                   jax.ShapeDtypeStruct((1, n_pad), jnp.float32)),
    )(x2, xp2)
    ei = jnp.pad(edge_index[:, 0], ((0, 0), (0, n_pad - n), (0, 0)))
    eii = _split_edges(ei[1], npw0, npw1, k)
    eij = _split_edges(ei[0], npw0, npw1, k)

    sc = _make_sc_kernel(n_pad, c, k, npw0, npw1)
    out = sc(xall, eii, eij, m2.reshape(n_pad), v2.reshape(n_pad))
    return out[:n].T[None, :, :, None]
